# baseline (speedup 1.0000x reference)
import functools
import math

import jax
import jax.numpy as jnp
from jax.experimental import pallas as pl

_BS = 4096


def _matvec_kernel(h_ref, w_ref, out_ref):
    out_ref[0] = jax.lax.dot_general(
        w_ref[...], h_ref[0],
        dimension_numbers=(((1,), (1,)), ((), ())),
        preferred_element_type=jnp.float32)


def _select_kernel(lg_ref, depth_ref, mask_ref, loss_ref, *, k):
    lg = lg_ref[...]
    b, _, s = lg.shape
    s0 = jax.nn.sigmoid(lg[:, 0, :])

    keys = jax.lax.bitcast_convert_type(s0, jnp.int32)
    t = jnp.zeros((b, 1), jnp.int32)
    for j in range(30, -1, -1):
        cand = t | (1 << j)
        cnt = jnp.sum((keys >= cand).astype(jnp.int32), axis=1, keepdims=True)
        t = jnp.where(cnt >= k, cand, t)

    gt = keys > t
    eq = keys == t
    cnt_gt = jnp.sum(gt.astype(jnp.int32), axis=1, keepdims=True)
    need = k - cnt_gt
    idx = jax.lax.broadcasted_iota(jnp.int32, (b, s), 1)
    m = jnp.zeros((b, 1), jnp.int32)
    for j in range(13, -1, -1):
        cand = m + (1 << j)
        cnt = jnp.sum((eq & (idx < cand)).astype(jnp.int32),
                      axis=1, keepdims=True)
        m = jnp.where((cand <= s) & (cnt <= need), cand, m)
    mask = gt | (eq & (idx < m))

    mask_ref[...] = mask
    depth_ref[...] = jnp.where(mask, 3, 1).astype(jnp.int32)

    sig2 = jax.nn.sigmoid(jax.nn.sigmoid(lg))
    inv = 1.0 / (b * s)
    one = jnp.ones((1, 1), jnp.float32)
    log_t = math.log(1.0 / 3.0)
    acc = one * (3.0 * log_t)
    for r in range(3):
        pr = jnp.sum(sig2[:, r, :]) * inv
        acc = acc - jnp.log(one * pr)
    loss_ref[...] = acc * (1.0 / 9.0)


def kernel(hidden_states, w0, w1, w2):
    b, s, h = hidden_states.shape
    k = max(1, int(s * (1.0 / 3.0)))
    w3 = jnp.stack([w0, w1, w2], axis=0)

    logits = pl.pallas_call(
        _matvec_kernel,
        grid=(b, s // _BS),
        in_specs=[
            pl.BlockSpec((1, _BS, h), lambda i, j: (i, j, 0)),
            pl.BlockSpec((3, h), lambda i, j: (0, 0)),
        ],
        out_specs=pl.BlockSpec((1, 3, _BS), lambda i, j: (i, 0, j)),
        out_shape=jax.ShapeDtypeStruct((b, 3, s), jnp.float32),
    )(hidden_states, w3)

    depth, mask, loss = pl.pallas_call(
        functools.partial(_select_kernel, k=k),
        out_shape=(
            jax.ShapeDtypeStruct((b, s), jnp.int32),
            jax.ShapeDtypeStruct((b, s), jnp.bool_),
            jax.ShapeDtypeStruct((1, 1), jnp.float32),
        ),
    )(logits)

    ones = jnp.ones((b, s), dtype=jnp.bool_)
    return (depth, loss[0, 0], ones, mask, mask)


# speedup vs baseline: 1.1790x; 1.1790x over previous
import functools
import math

import jax
import jax.numpy as jnp
from jax.experimental import pallas as pl

_BS = 2048


def _matvec_kernel(h_ref, w_ref, lg0_ref, sums_ref):
    lg = jax.lax.dot_general(
        w_ref[...], h_ref[0],
        dimension_numbers=(((1,), (1,)), ((), ())),
        preferred_element_type=jnp.float32)
    lg0_ref[0, 0, 0] = lg[0]
    part = jax.nn.sigmoid(jax.nn.sigmoid(lg))
    step = pl.program_id(0) * pl.num_programs(1) + pl.program_id(1)

    @pl.when(step == 0)
    def _init():
        sums_ref[...] = part

    @pl.when(step != 0)
    def _acc():
        sums_ref[...] += part


def _select_kernel(lg0_ref, sums_ref, depth_ref, mask_ref, loss_ref, *, k):
    lg0 = lg0_ref[...]
    b, r8, c = lg0.shape
    s = r8 * c
    s0 = jax.nn.sigmoid(lg0)

    keys = jax.lax.bitcast_convert_type(s0, jnp.int32)
    t = jnp.zeros((b, 1, 1), jnp.int32)
    for j in range(29, -1, -1):
        cand = t | (1 << j)
        cnt = jnp.sum((keys >= cand).astype(jnp.int32),
                      axis=(1, 2), keepdims=True)
        t = jnp.where(cnt >= k, cand, t)

    gt = keys > t
    eq = keys == t
    cnt_gt = jnp.sum(gt.astype(jnp.int32), axis=(1, 2), keepdims=True)
    need = k - cnt_gt
    idx = (jax.lax.broadcasted_iota(jnp.int32, (b, r8, c), 1) * c
           + jax.lax.broadcasted_iota(jnp.int32, (b, r8, c), 2))
    m = jnp.zeros((b, 1, 1), jnp.int32)
    for j in range(13, -1, -1):
        cand = m + (1 << j)
        cnt = jnp.sum((eq & (idx < cand)).astype(jnp.int32),
                      axis=(1, 2), keepdims=True)
        m = jnp.where((cand <= s) & (cnt <= need), cand, m)
    mask = gt | (eq & (idx < m))

    mask_ref[...] = mask
    depth_ref[...] = jnp.where(mask, 3, 1).astype(jnp.int32)

    inv = 1.0 / (b * s)
    one = jnp.ones((1, 1), jnp.float32)
    log_t = math.log(1.0 / 3.0)
    acc = one * (3.0 * log_t)
    for r in range(3):
        pr = jnp.sum(sums_ref[r, :]) * inv
        acc = acc - jnp.log(one * pr)
    loss_ref[...] = acc * (1.0 / 9.0)


def kernel(hidden_states, w0, w1, w2):
    b, s, h = hidden_states.shape
    k = max(1, int(s * (1.0 / 3.0)))
    w3 = jnp.stack([w0, w1, w2], axis=0)
    nb = s // _BS

    lg0, sums = pl.pallas_call(
        _matvec_kernel,
        grid=(b, nb),
        in_specs=[
            pl.BlockSpec((1, _BS, h), lambda i, j: (i, j, 0)),
            pl.BlockSpec((3, h), lambda i, j: (0, 0)),
        ],
        out_specs=[
            pl.BlockSpec((1, 1, 1, _BS), lambda i, j: (i, j, 0, 0)),
            pl.BlockSpec((3, _BS), lambda i, j: (0, 0)),
        ],
        out_shape=[
            jax.ShapeDtypeStruct((b, nb, 1, _BS), jnp.float32),
            jax.ShapeDtypeStruct((3, _BS), jnp.float32),
        ],
    )(hidden_states, w3)

    lg0 = lg0.reshape(b, 8, s // 8)
    depth, mask, loss = pl.pallas_call(
        functools.partial(_select_kernel, k=k),
        out_shape=(
            jax.ShapeDtypeStruct((b, 8, s // 8), jnp.int32),
            jax.ShapeDtypeStruct((b, 8, s // 8), jnp.bool_),
            jax.ShapeDtypeStruct((1, 1), jnp.float32),
        ),
    )(lg0, sums)

    ones = jnp.ones((b, s), dtype=jnp.bool_)
    return (depth.reshape(b, s), loss[0, 0], ones,
            mask.reshape(b, s), mask.reshape(b, s))
